# Initial kernel scaffold; baseline (speedup 1.0000x reference)
import jax
import jax.numpy as jnp
from jax.experimental import pallas as pl


def kernel(head, relation, tail, entity_emb, relation_emb):
    raise NotImplementedError("write your pallas kernel here")


# speedup vs baseline: 2.3039x; 2.3039x over previous
import functools

import jax
import jax.numpy as jnp
from jax import lax
from jax.experimental import pallas as pl
from jax.experimental.pallas import tpu as pltpu
from jax.experimental.pallas import tpu_sc as plsc

B = 16384
D = 128
LANES = 16
NUM_CORES = 2
NUM_SUBCORES = 16
NW = NUM_CORES * NUM_SUBCORES
BPW = B // NW
CHUNK = 128
NCHUNK = BPW // CHUNK


def _sc_kernel(head_hbm, rel_hbm, tail_hbm, ent_hbm, relemb_hbm, out_hbm,
               hidx, ridx, tidx, hbuf, rbuf, tbuf, scores,
               semh, semr, semt):
    wid = lax.axis_index("s") * NUM_CORES + lax.axis_index("c")
    base = wid * BPW

    pltpu.sync_copy(head_hbm.at[pl.ds(base, BPW)], hidx)
    pltpu.sync_copy(rel_hbm.at[pl.ds(base, BPW)], ridx)
    pltpu.sync_copy(tail_hbm.at[pl.ds(base, BPW)], tidx)

    def chunk_body(ci, carry):
        cb = ci * CHUNK
        ch = pltpu.async_copy(ent_hbm.at[hidx.at[pl.ds(cb, CHUNK)]], hbuf, semh)
        cr = pltpu.async_copy(relemb_hbm.at[ridx.at[pl.ds(cb, CHUNK)]], rbuf, semr)
        ct = pltpu.async_copy(ent_hbm.at[tidx.at[pl.ds(cb, CHUNK)]], tbuf, semt)
        ch.wait()
        cr.wait()
        ct.wait()

        lane = lax.iota(jnp.int32, LANES)

        def group_body(g, c2):
            gb = g * LANES

            def row_body(j, vec):
                i = gb + j
                acc = (hbuf[i, pl.ds(0, LANES)]
                       * rbuf[i, pl.ds(0, LANES)]
                       * tbuf[i, pl.ds(0, LANES)])
                for k in range(1, D // LANES):
                    o = k * LANES
                    acc = acc + (hbuf[i, pl.ds(o, LANES)]
                                 * rbuf[i, pl.ds(o, LANES)]
                                 * tbuf[i, pl.ds(o, LANES)])
                for s in (8, 4, 2, 1):
                    acc = acc + jnp.take_along_axis(
                        acc, lane ^ s, axis=0, mode="promise_in_bounds")
                return jnp.where(lane == j, acc, vec)

            vec = lax.fori_loop(0, LANES, row_body,
                                jnp.zeros((LANES,), jnp.float32))
            scores[pl.ds(cb + gb, LANES)] = vec
            return c2

        return lax.fori_loop(0, CHUNK // LANES, group_body, carry)

    lax.fori_loop(0, NCHUNK, chunk_body, 0)
    pltpu.sync_copy(scores, out_hbm.at[pl.ds(base, BPW)])


@functools.partial(
    pl.kernel,
    mesh=plsc.VectorSubcoreMesh(core_axis_name="c", subcore_axis_name="s"),
    out_type=jax.ShapeDtypeStruct((B,), jnp.float32),
    scratch_types=[
        pltpu.VMEM((BPW,), jnp.int32),
        pltpu.VMEM((BPW,), jnp.int32),
        pltpu.VMEM((BPW,), jnp.int32),
        pltpu.VMEM((CHUNK, D), jnp.float32),
        pltpu.VMEM((CHUNK, D), jnp.float32),
        pltpu.VMEM((CHUNK, D), jnp.float32),
        pltpu.VMEM((BPW,), jnp.float32),
        pltpu.SemaphoreType.DMA,
        pltpu.SemaphoreType.DMA,
        pltpu.SemaphoreType.DMA,
    ],
)
def _distmult_sc(head_hbm, rel_hbm, tail_hbm, ent_hbm, relemb_hbm, out_hbm,
                 hidx, ridx, tidx, hbuf, rbuf, tbuf, scores,
                 semh, semr, semt):
    _sc_kernel(head_hbm, rel_hbm, tail_hbm, ent_hbm, relemb_hbm, out_hbm,
               hidx, ridx, tidx, hbuf, rbuf, tbuf, scores,
               semh, semr, semt)


def kernel(head, relation, tail, entity_emb, relation_emb):
    head = head.astype(jnp.int32)
    relation = relation.astype(jnp.int32)
    tail = tail.astype(jnp.int32)
    return _distmult_sc(head, relation, tail, entity_emb, relation_emb)
